# baseline (speedup 1.0000x reference)
import functools

import jax
import jax.numpy as jnp
from jax import lax
from jax.experimental import pallas as pl
from jax.experimental.pallas import tpu as pltpu
from jax.experimental.pallas import tpu_sc as plsc

N = 10000
E = 160000
IN_DIM = 256
HID = 512
OUT_DIM = 256

NC = 2
NS = 16
NW = NC * NS
L = 16

RB = 400
GRID = N // RB

EPW = E // NW
DEG_BATCHES = (EPW + L - 1) // L
EPW_PAD = DEG_BATCHES * L


def _deg_body(row_hbm, parts_hbm, rowv, degv):
    wid = lax.axis_index("s") * NC + lax.axis_index("c")
    pltpu.sync_copy(row_hbm.at[pl.ds(wid * EPW, EPW)], rowv.at[pl.ds(0, EPW)])

    def zero_step(i, _):
        degv[pl.ds(i * L, L)] = jnp.zeros((L,), jnp.float32)
        return 0

    lax.fori_loop(0, N // L, zero_step, 0)

    ones = jnp.ones((L,), jnp.float32)
    lane = lax.iota(jnp.int32, L)

    def step(j, _):
        idx = rowv[pl.ds(j * L, L)]
        mask = (j * L + lane) < EPW
        plsc.addupdate_scatter(degv, [idx], ones, mask=mask)
        return 0

    lax.fori_loop(0, DEG_BATCHES, step, 0)
    pltpu.sync_copy(degv, parts_hbm.at[pl.ds(wid * N, N)])


@functools.lru_cache(maxsize=None)
def _deg_kernel():
    return pl.kernel(
        _deg_body,
        out_type=jax.ShapeDtypeStruct((NW * N,), jnp.float32),
        mesh=plsc.VectorSubcoreMesh(
            core_axis_name="c", subcore_axis_name="s",
            num_cores=NC, num_subcores=NS),
        scratch_types=[
            pltpu.VMEM((EPW_PAD,), jnp.int32),
            pltpu.VMEM((N,), jnp.float32),
        ],
        compiler_params=pltpu.CompilerParams(needs_layout_passes=False),
    )


def _deg_call(row_flat):
    return _deg_kernel()(row_flat).reshape(NW, N)

K = 80
EPS = E // NS
NB = EPS // K
NP = 10240
RPT = NP // NS
ZR = 40


def _spmm_body(cpc, hs_hbm, rows_hbm, cols_hbm, agg_hbm,
               rowv, colv, rowb0, rowb1, colb0, colb1, gbuf0, gbuf1,
               zbuf, acc, sem0, sem1, ssem0, ssem1):
    cid = lax.axis_index("c")
    sid = lax.axis_index("s")
    pltpu.sync_copy(rows_hbm.at[pl.ds(sid * EPS, EPS)], rowv)
    pltpu.sync_copy(cols_hbm.at[pl.ds(sid * EPS, EPS)], colv)

    gk = K // L

    def zero_step(t, _):
        r = t // (128 // L)
        q = t % (128 // L)
        zbuf[r, pl.ds(q * L, L)] = jnp.zeros((L,), jnp.float32)
        return 0

    lax.fori_loop(0, ZR * (128 // L), zero_step, 0)

    def start_gather(j, offv, rowb, colb, gbuf, sem):
        def stage(i, _):
            colb[pl.ds(i * L, L)] = colv[pl.ds(j * K + i * L, L)] + offv
            rowb[pl.ds(i * L, L)] = rowv[pl.ds(j * K + i * L, L)]
            return 0

        lax.fori_loop(0, gk, stage, 0)
        pltpu.async_copy(hs_hbm.at[colb], gbuf, sem)

    def wait_gather(gbuf, sem):
        pltpu.make_async_copy(hs_hbm.at[pl.ds(0, K)], gbuf, sem).wait()

    def start_scatter(rowb, gbuf, ssem):
        pltpu.async_copy(gbuf, acc.at[rowb], ssem, add=True)

    def wait_scatter(gbuf, ssem):
        pltpu.make_async_copy(hs_hbm.at[pl.ds(0, K)], gbuf, ssem).wait()

    for l in range(cpc):
        chunk = cid * cpc + l
        offv = chunk * NP

        def zacc_step(z, _):
            pltpu.sync_copy(zbuf, acc.at[pl.ds(sid * RPT + z * ZR, ZR)])
            return 0

        lax.fori_loop(0, RPT // ZR, zacc_step, 0)
        plsc.subcore_barrier()

        start_gather(0, offv, rowb0, colb0, gbuf0, sem0)

        def batch_step(j, _):
            @pl.when(j >= 1)
            def _():
                @pl.when(j % 2 == 1)
                def _():
                    wait_scatter(gbuf0, ssem0)

                @pl.when(j % 2 == 0)
                def _():
                    wait_scatter(gbuf1, ssem1)

            @pl.when(j + 1 < NB)
            def _():
                @pl.when((j + 1) % 2 == 0)
                def _():
                    start_gather(j + 1, offv, rowb0, colb0, gbuf0, sem0)

                @pl.when((j + 1) % 2 == 1)
                def _():
                    start_gather(j + 1, offv, rowb1, colb1, gbuf1, sem1)

            @pl.when(j % 2 == 0)
            def _():
                wait_gather(gbuf0, sem0)
                start_scatter(rowb0, gbuf0, ssem0)

            @pl.when(j % 2 == 1)
            def _():
                wait_gather(gbuf1, sem1)
                start_scatter(rowb1, gbuf1, ssem1)

            return 0

        lax.fori_loop(0, NB, batch_step, 0)

        @pl.when((NB - 1) % 2 == 0)
        def _():
            wait_scatter(gbuf0, ssem0)

        @pl.when((NB - 1) % 2 == 1)
        def _():
            wait_scatter(gbuf1, ssem1)

        plsc.subcore_barrier()
        pltpu.sync_copy(
            acc.at[pl.ds(sid * RPT, RPT)],
            agg_hbm.at[pl.ds(offv + sid * RPT, RPT)],
        )
        plsc.subcore_barrier()


def _svec_body(dinv_hbm, rows_hbm, cols_hbm, sparts_hbm,
               rowv, colv, dinvv, sacc):
    wid = lax.axis_index("s") * NC + lax.axis_index("c")
    rowv[pl.ds(EPW_PAD - L, L)] = jnp.zeros((L,), jnp.int32)
    colv[pl.ds(EPW_PAD - L, L)] = jnp.zeros((L,), jnp.int32)
    pltpu.sync_copy(rows_hbm.at[pl.ds(wid * EPW, EPW)], rowv.at[pl.ds(0, EPW)])
    pltpu.sync_copy(cols_hbm.at[pl.ds(wid * EPW, EPW)], colv.at[pl.ds(0, EPW)])
    pltpu.sync_copy(dinv_hbm, dinvv)

    def zero_step(i, _):
        sacc[pl.ds(i * L, L)] = jnp.zeros((L,), jnp.float32)
        return 0

    lax.fori_loop(0, N // L, zero_step, 0)

    lane = lax.iota(jnp.int32, L)

    def step(j, _):
        idx_c = colv[pl.ds(j * L, L)]
        idx_r = rowv[pl.ds(j * L, L)]
        mask = (j * L + lane) < EPW
        val = plsc.load_gather(dinvv, [idx_c], mask=mask)
        plsc.addupdate_scatter(sacc, [idx_r], val, mask=mask)
        return 0

    lax.fori_loop(0, DEG_BATCHES, step, 0)
    pltpu.sync_copy(sacc, sparts_hbm.at[pl.ds(wid * N, N)])


@functools.lru_cache(maxsize=None)
def _svec_kernel():
    return pl.kernel(
        _svec_body,
        out_type=jax.ShapeDtypeStruct((NW * N,), jnp.float32),
        mesh=plsc.VectorSubcoreMesh(
            core_axis_name="c", subcore_axis_name="s",
            num_cores=NC, num_subcores=NS),
        scratch_types=[
            pltpu.VMEM((EPW_PAD,), jnp.int32),
            pltpu.VMEM((EPW_PAD,), jnp.int32),
            pltpu.VMEM((N,), jnp.float32),
            pltpu.VMEM((N,), jnp.float32),
        ],
        compiler_params=pltpu.CompilerParams(needs_layout_passes=False),
    )


def _svec_call(dinv_vec, row, col):
    return _svec_kernel()(dinv_vec, row, col).reshape(NW, N)


@functools.lru_cache(maxsize=None)
def _make_spmm(c_chunks):
    cpc = c_chunks // NC
    return pl.kernel(
        functools.partial(_spmm_body, cpc),
        out_type=jax.ShapeDtypeStruct((c_chunks * NP, 128), jnp.float32),
        mesh=plsc.VectorSubcoreMesh(
            core_axis_name="c", subcore_axis_name="s",
            num_cores=NC, num_subcores=NS),
        scratch_types=[
            pltpu.VMEM((EPS,), jnp.int32),
            pltpu.VMEM((EPS,), jnp.int32),
            pltpu.VMEM((K,), jnp.int32),
            pltpu.VMEM((K,), jnp.int32),
            pltpu.VMEM((K,), jnp.int32),
            pltpu.VMEM((K,), jnp.int32),
            pltpu.VMEM((K, 128), jnp.float32),
            pltpu.VMEM((K, 128), jnp.float32),
            pltpu.VMEM((ZR, 128), jnp.float32),
            pltpu.VMEM_SHARED((NP, 128), jnp.float32),
            pltpu.SemaphoreType.DMA,
            pltpu.SemaphoreType.DMA,
            pltpu.SemaphoreType.DMA,
            pltpu.SemaphoreType.DMA,
        ],
        compiler_params=pltpu.CompilerParams(needs_layout_passes=False),
    )


def _spmm4(hs_flat, row, col):
    return _make_spmm(4)(hs_flat, row, col)


def _spmm2(hs_flat, row, col):
    return _make_spmm(2)(hs_flat, row, col)


def _tc_dinv_body(parts_ref, dinv_ref):
    deg = jnp.sum(parts_ref[...], axis=0)
    dinv_ref[...] = jnp.where(
        deg > 0, lax.rsqrt(jnp.maximum(deg, 1e-12)), 0.0)


def _tc_dinv(parts):
    return pl.pallas_call(
        _tc_dinv_body,
        out_shape=jax.ShapeDtypeStruct((N,), jnp.float32),
    )(parts).reshape(N, 1)


def _tc_first_body(dinv_ref, x_ref, w_ref, b_ref, h_ref, xsf_ref):
    dinv = dinv_ref[...]
    h = jnp.dot(x_ref[...], w_ref[...], preferred_element_type=jnp.float32)
    h_ref[...] = h + b_ref[...]
    xs = x_ref[...] * dinv
    for c in range(xsf_ref.shape[0]):
        xsf_ref[c] = xs[:, c * 128:(c + 1) * 128]


def _tc_svec_body(sparts_ref, dinv_ref, s_ref):
    sp = jnp.sum(sparts_ref[...], axis=0)
    s_ref[...] = sp * dinv_ref[...][:, 0]


def _tc_svec(sparts, dinv):
    return pl.pallas_call(
        _tc_svec_body,
        out_shape=jax.ShapeDtypeStruct((N,), jnp.float32),
    )(sparts, dinv).reshape(N, 1)


def _tc_layer2_body(dinv_ref, s_ref, h1_ref, aggxf_ref, w1_ref, b1_ref,
                    w2_ref, b2_ref, ho_ref, hsf_ref):
    dinv = dinv_ref[...]
    aggx = jnp.concatenate(
        [aggxf_ref[c] for c in range(aggxf_ref.shape[0])], axis=1)
    corr = jnp.dot(dinv * aggx, w1_ref[...],
                   preferred_element_type=jnp.float32)
    t = jnp.maximum(h1_ref[...] - corr - s_ref[...] * b1_ref[...], 0.0)
    h = jnp.dot(t, w2_ref[...], preferred_element_type=jnp.float32)
    h = h + b2_ref[...]
    ho_ref[...] = h
    hs = h * dinv
    for c in range(hsf_ref.shape[0]):
        hsf_ref[c] = hs[:, c * 128:(c + 1) * 128]


def _tc_layer2(dinv, svec, h1, aggxf, w1, b1, w2, b2, cout):
    d1_in, d1_out = w1.shape
    d2_out = w2.shape[1]
    cin = aggxf.shape[0]
    return pl.pallas_call(
        _tc_layer2_body,
        grid=(GRID,),
        in_specs=[
            _dinv_spec(), _dinv_spec(), _rows_spec(d1_out), _chunk_spec(cin),
            _full_spec(d1_in, d1_out), _full_spec(1, d1_out),
            _full_spec(d1_out, d2_out), _full_spec(1, d2_out),
        ],
        out_specs=[_rows_spec(d2_out), _chunk_spec(cout)],
        out_shape=[
            jax.ShapeDtypeStruct((N, d2_out), jnp.float32),
            jax.ShapeDtypeStruct((cout, NP, 128), jnp.float32),
        ],
    )(dinv, svec, h1, aggxf, w1, b1, w2, b2)


def _tc_mid_body(dinv_ref, h_ref, aggf_ref, w_ref, b_ref, ho_ref, hsf_ref):
    dinv = dinv_ref[...]
    agg = jnp.concatenate(
        [aggf_ref[c] for c in range(aggf_ref.shape[0])], axis=1)
    t = jnp.maximum(h_ref[...] - dinv * agg, 0.0)
    h = jnp.dot(t, w_ref[...], preferred_element_type=jnp.float32)
    h = h + b_ref[...]
    ho_ref[...] = h
    hs = h * dinv
    for c in range(hsf_ref.shape[0]):
        hsf_ref[c] = hs[:, c * 128:(c + 1) * 128]


def _tc_last_body(dinv_ref, h_ref, aggf_ref, out_ref):
    dinv = dinv_ref[...]
    agg = jnp.concatenate(
        [aggf_ref[c] for c in range(aggf_ref.shape[0])], axis=1)
    out_ref[...] = h_ref[...] - dinv * agg


def _dinv_spec():
    return pl.BlockSpec((RB, 1), lambda i: (i, 0))


def _rows_spec(d):
    return pl.BlockSpec((RB, d), lambda i: (i, 0))


def _full_spec(r, c):
    return pl.BlockSpec((r, c), lambda i: (0, 0))


def _chunk_spec(c):
    return pl.BlockSpec((c, RB, 128), lambda i: (0, i, 0))


def _tc_first(dinv, x, w, b, cout):
    d_in, d_out = w.shape
    return pl.pallas_call(
        _tc_first_body,
        grid=(GRID,),
        in_specs=[
            _dinv_spec(), _rows_spec(d_in),
            _full_spec(d_in, d_out), _full_spec(1, d_out),
        ],
        out_specs=[_rows_spec(d_out), _chunk_spec(cout)],
        out_shape=[
            jax.ShapeDtypeStruct((N, d_out), jnp.float32),
            jax.ShapeDtypeStruct((cout, NP, 128), jnp.float32),
        ],
    )(dinv, x, w, b)


def _tc_mid(dinv, h, aggf, w, b, cout):
    d_in, d_out = w.shape
    cin = aggf.shape[0]
    return pl.pallas_call(
        _tc_mid_body,
        grid=(GRID,),
        in_specs=[
            _dinv_spec(), _rows_spec(d_in), _chunk_spec(cin),
            _full_spec(d_in, d_out), _full_spec(1, d_out),
        ],
        out_specs=[_rows_spec(d_out), _chunk_spec(cout)],
        out_shape=[
            jax.ShapeDtypeStruct((N, d_out), jnp.float32),
            jax.ShapeDtypeStruct((cout, NP, 128), jnp.float32),
        ],
    )(dinv, h, aggf, w, b)


def _tc_last(dinv, h, aggf):
    d = h.shape[1]
    cin = aggf.shape[0]
    return pl.pallas_call(
        _tc_last_body,
        grid=(GRID,),
        in_specs=[_dinv_spec(), _rows_spec(d), _chunk_spec(cin)],
        out_specs=_rows_spec(d),
        out_shape=jax.ShapeDtypeStruct((N, d), jnp.float32),
    )(dinv, h, aggf)


def kernel(x, edge_index, W1, b1, W2, b2, W3, b3):
    row = edge_index[0]
    col = edge_index[1]

    parts = _deg_call(row)
    dinv = _tc_dinv(parts)

    h1, xsf = _tc_first(dinv, x, W1, b1.reshape(1, -1), IN_DIM // 128)
    aggx = _make_spmm(2)(xsf.reshape(-1, 128), row, col)
    sparts = _svec_call(dinv.reshape(-1), row, col)
    svec = _tc_svec(sparts, dinv)

    h2, hs2 = _tc_layer2(
        dinv, svec, h1, aggx.reshape(-1, NP, 128),
        W1, b1.reshape(1, -1), W2, b2.reshape(1, -1), HID // 128)
    agg2 = _spmm4(hs2.reshape(-1, 128), row, col).reshape(-1, NP, 128)

    h3, hs3 = _tc_mid(dinv, h2, agg2, W3, b3.reshape(1, -1), OUT_DIM // 128)
    agg3 = _spmm2(hs3.reshape(-1, 128), row, col).reshape(-1, NP, 128)

    return _tc_last(dinv, h3, agg3)


# speedup vs baseline: 1.0075x; 1.0075x over previous
import functools

import jax
import jax.numpy as jnp
from jax import lax
from jax.experimental import pallas as pl
from jax.experimental.pallas import tpu as pltpu
from jax.experimental.pallas import tpu_sc as plsc

N = 10000
E = 160000
IN_DIM = 256
HID = 512
OUT_DIM = 256

NC = 2
NS = 16
NW = NC * NS
L = 16

RB = 400
GRID = N // RB

EPW = E // NW
DEG_BATCHES = (EPW + L - 1) // L
EPW_PAD = DEG_BATCHES * L


def _deg_body(row_hbm, parts_hbm, rowv, degv):
    wid = lax.axis_index("s") * NC + lax.axis_index("c")
    pltpu.sync_copy(row_hbm.at[pl.ds(wid * EPW, EPW)], rowv.at[pl.ds(0, EPW)])

    def zero_step(i, _):
        degv[pl.ds(i * L, L)] = jnp.zeros((L,), jnp.float32)
        return 0

    lax.fori_loop(0, N // L, zero_step, 0)

    ones = jnp.ones((L,), jnp.float32)
    lane = lax.iota(jnp.int32, L)

    def step(j, _):
        idx = rowv[pl.ds(j * L, L)]
        mask = (j * L + lane) < EPW
        plsc.addupdate_scatter(degv, [idx], ones, mask=mask)
        return 0

    lax.fori_loop(0, DEG_BATCHES, step, 0)
    pltpu.sync_copy(degv, parts_hbm.at[pl.ds(wid * N, N)])


@functools.lru_cache(maxsize=None)
def _deg_kernel():
    return pl.kernel(
        _deg_body,
        out_type=jax.ShapeDtypeStruct((NW * N,), jnp.float32),
        mesh=plsc.VectorSubcoreMesh(
            core_axis_name="c", subcore_axis_name="s",
            num_cores=NC, num_subcores=NS),
        scratch_types=[
            pltpu.VMEM((EPW_PAD,), jnp.int32),
            pltpu.VMEM((N,), jnp.float32),
        ],
        compiler_params=pltpu.CompilerParams(needs_layout_passes=False),
    )


def _deg_call(row_flat):
    return _deg_kernel()(row_flat).reshape(NW, N)

K = 80
EPS = E // NS
NB = EPS // K
NP = 10240
RPT = NP // NS
ZR = 40


def _spmm_body(cpc, hs_hbm, rows_hbm, cols_hbm, agg_hbm,
               rowv, colv, rowb0, rowb1, colb0, colb1, gbuf0, gbuf1,
               zbuf, acc, sem0, sem1):
    cid = lax.axis_index("c")
    sid = lax.axis_index("s")
    pltpu.sync_copy(rows_hbm.at[pl.ds(sid * EPS, EPS)], rowv)
    pltpu.sync_copy(cols_hbm.at[pl.ds(sid * EPS, EPS)], colv)

    def zero_step(t, _):
        r = t // (128 // L)
        q = t % (128 // L)
        zbuf[r, pl.ds(q * L, L)] = jnp.zeros((L,), jnp.float32)
        return 0

    lax.fori_loop(0, ZR * (128 // L), zero_step, 0)

    def start_gather(j, offv, rowb, colb, gbuf, sem):
        def stage(i, _):
            colb[pl.ds(i * L, L)] = colv[pl.ds(j * K + i * L, L)] + offv
            rowb[pl.ds(i * L, L)] = rowv[pl.ds(j * K + i * L, L)]
            return 0

        lax.fori_loop(0, K // L, stage, 0)
        pltpu.async_copy(hs_hbm.at[colb], gbuf, sem)

    def finish_batch(rowb, gbuf, sem):
        pltpu.make_async_copy(hs_hbm.at[pl.ds(0, K)], gbuf, sem).wait()
        pltpu.sync_copy(gbuf, acc.at[rowb], add=True)

    for l in range(cpc):
        chunk = cid * cpc + l
        offv = chunk * NP

        def zacc_step(z, _):
            pltpu.sync_copy(zbuf, acc.at[pl.ds(sid * RPT + z * ZR, ZR)])
            return 0

        lax.fori_loop(0, RPT // ZR, zacc_step, 0)
        plsc.subcore_barrier()

        start_gather(0, offv, rowb0, colb0, gbuf0, sem0)

        def batch_step(j, _):
            nxt = j + 1

            @pl.when(nxt < NB)
            def _():
                @pl.when(nxt % 2 == 0)
                def _():
                    start_gather(nxt, offv, rowb0, colb0, gbuf0, sem0)

                @pl.when(nxt % 2 == 1)
                def _():
                    start_gather(nxt, offv, rowb1, colb1, gbuf1, sem1)

            @pl.when(j % 2 == 0)
            def _():
                finish_batch(rowb0, gbuf0, sem0)

            @pl.when(j % 2 == 1)
            def _():
                finish_batch(rowb1, gbuf1, sem1)

            return 0

        lax.fori_loop(0, NB, batch_step, 0)
        plsc.subcore_barrier()
        pltpu.sync_copy(
            acc.at[pl.ds(sid * RPT, RPT)],
            agg_hbm.at[pl.ds(offv + sid * RPT, RPT)],
        )
        plsc.subcore_barrier()


def _svec_body(dinv_hbm, rows_hbm, cols_hbm, sparts_hbm,
               rowv, colv, dinvv, sacc):
    wid = lax.axis_index("s") * NC + lax.axis_index("c")
    rowv[pl.ds(EPW_PAD - L, L)] = jnp.zeros((L,), jnp.int32)
    colv[pl.ds(EPW_PAD - L, L)] = jnp.zeros((L,), jnp.int32)
    pltpu.sync_copy(rows_hbm.at[pl.ds(wid * EPW, EPW)], rowv.at[pl.ds(0, EPW)])
    pltpu.sync_copy(cols_hbm.at[pl.ds(wid * EPW, EPW)], colv.at[pl.ds(0, EPW)])
    pltpu.sync_copy(dinv_hbm, dinvv)

    def zero_step(i, _):
        sacc[pl.ds(i * L, L)] = jnp.zeros((L,), jnp.float32)
        return 0

    lax.fori_loop(0, N // L, zero_step, 0)

    lane = lax.iota(jnp.int32, L)

    def step(j, _):
        idx_c = colv[pl.ds(j * L, L)]
        idx_r = rowv[pl.ds(j * L, L)]
        mask = (j * L + lane) < EPW
        val = plsc.load_gather(dinvv, [idx_c], mask=mask)
        plsc.addupdate_scatter(sacc, [idx_r], val, mask=mask)
        return 0

    lax.fori_loop(0, DEG_BATCHES, step, 0)
    pltpu.sync_copy(sacc, sparts_hbm.at[pl.ds(wid * N, N)])


@functools.lru_cache(maxsize=None)
def _svec_kernel():
    return pl.kernel(
        _svec_body,
        out_type=jax.ShapeDtypeStruct((NW * N,), jnp.float32),
        mesh=plsc.VectorSubcoreMesh(
            core_axis_name="c", subcore_axis_name="s",
            num_cores=NC, num_subcores=NS),
        scratch_types=[
            pltpu.VMEM((EPW_PAD,), jnp.int32),
            pltpu.VMEM((EPW_PAD,), jnp.int32),
            pltpu.VMEM((N,), jnp.float32),
            pltpu.VMEM((N,), jnp.float32),
        ],
        compiler_params=pltpu.CompilerParams(needs_layout_passes=False),
    )


def _svec_call(dinv_vec, row, col):
    return _svec_kernel()(dinv_vec, row, col).reshape(NW, N)


@functools.lru_cache(maxsize=None)
def _make_spmm(c_chunks):
    cpc = c_chunks // NC
    return pl.kernel(
        functools.partial(_spmm_body, cpc),
        out_type=jax.ShapeDtypeStruct((c_chunks * NP, 128), jnp.float32),
        mesh=plsc.VectorSubcoreMesh(
            core_axis_name="c", subcore_axis_name="s",
            num_cores=NC, num_subcores=NS),
        scratch_types=[
            pltpu.VMEM((EPS,), jnp.int32),
            pltpu.VMEM((EPS,), jnp.int32),
            pltpu.VMEM((K,), jnp.int32),
            pltpu.VMEM((K,), jnp.int32),
            pltpu.VMEM((K,), jnp.int32),
            pltpu.VMEM((K,), jnp.int32),
            pltpu.VMEM((K, 128), jnp.float32),
            pltpu.VMEM((K, 128), jnp.float32),
            pltpu.VMEM((ZR, 128), jnp.float32),
            pltpu.VMEM_SHARED((NP, 128), jnp.float32),
            pltpu.SemaphoreType.DMA,
            pltpu.SemaphoreType.DMA,
        ],
        compiler_params=pltpu.CompilerParams(needs_layout_passes=False),
    )


def _spmm4(hs_flat, row, col):
    return _make_spmm(4)(hs_flat, row, col)


def _spmm2(hs_flat, row, col):
    return _make_spmm(2)(hs_flat, row, col)


def _tc_dinv_body(parts_ref, dinv_ref):
    deg = jnp.sum(parts_ref[...], axis=0)
    dinv_ref[...] = jnp.where(
        deg > 0, lax.rsqrt(jnp.maximum(deg, 1e-12)), 0.0)


def _tc_dinv(parts):
    return pl.pallas_call(
        _tc_dinv_body,
        out_shape=jax.ShapeDtypeStruct((N,), jnp.float32),
    )(parts).reshape(N, 1)


def _tc_first_body(dinv_ref, x_ref, w_ref, b_ref, h_ref, xsf_ref):
    dinv = dinv_ref[...]
    h = jnp.dot(x_ref[...].astype(jnp.bfloat16), w_ref[...],
                preferred_element_type=jnp.float32)
    h_ref[...] = h + b_ref[...]
    xs = x_ref[...] * dinv
    for c in range(xsf_ref.shape[0]):
        xsf_ref[c] = xs[:, c * 128:(c + 1) * 128]


def _tc_svec_body(sparts_ref, dinv_ref, s_ref):
    sp = jnp.sum(sparts_ref[...], axis=0)
    s_ref[...] = sp * dinv_ref[...][:, 0]


def _tc_svec(sparts, dinv):
    return pl.pallas_call(
        _tc_svec_body,
        out_shape=jax.ShapeDtypeStruct((N,), jnp.float32),
    )(sparts, dinv).reshape(N, 1)


def _tc_layer2_body(dinv_ref, s_ref, h1_ref, aggxf_ref, w1_ref, b1_ref,
                    w2_ref, b2_ref, ho_ref, hsf_ref):
    dinv = dinv_ref[...]
    aggx = jnp.concatenate(
        [aggxf_ref[c] for c in range(aggxf_ref.shape[0])], axis=1)
    corr = jnp.dot((dinv * aggx).astype(jnp.bfloat16), w1_ref[...],
                   preferred_element_type=jnp.float32)
    t = jnp.maximum(h1_ref[...] - corr - s_ref[...] * b1_ref[...], 0.0)
    h = jnp.dot(t.astype(jnp.bfloat16), w2_ref[...],
                preferred_element_type=jnp.float32)
    h = h + b2_ref[...]
    ho_ref[...] = h
    hs = h * dinv
    for c in range(hsf_ref.shape[0]):
        hsf_ref[c] = hs[:, c * 128:(c + 1) * 128]


def _tc_layer2(dinv, svec, h1, aggxf, w1, b1, w2, b2, cout):
    d1_in, d1_out = w1.shape
    d2_out = w2.shape[1]
    cin = aggxf.shape[0]
    return pl.pallas_call(
        _tc_layer2_body,
        grid=(GRID,),
        in_specs=[
            _dinv_spec(), _dinv_spec(), _rows_spec(d1_out), _chunk_spec(cin),
            _full_spec(d1_in, d1_out), _full_spec(1, d1_out),
            _full_spec(d1_out, d2_out), _full_spec(1, d2_out),
        ],
        out_specs=[_rows_spec(d2_out), _chunk_spec(cout)],
        out_shape=[
            jax.ShapeDtypeStruct((N, d2_out), jnp.float32),
            jax.ShapeDtypeStruct((cout, NP, 128), jnp.float32),
        ],
    )(dinv, svec, h1, aggxf, w1, b1, w2, b2)


def _tc_mid_body(dinv_ref, h_ref, aggf_ref, w_ref, b_ref, ho_ref, hsf_ref):
    dinv = dinv_ref[...]
    agg = jnp.concatenate(
        [aggf_ref[c] for c in range(aggf_ref.shape[0])], axis=1)
    t = jnp.maximum(h_ref[...] - dinv * agg, 0.0)
    h = jnp.dot(t.astype(jnp.bfloat16), w_ref[...],
                preferred_element_type=jnp.float32)
    h = h + b_ref[...]
    ho_ref[...] = h
    hs = h * dinv
    for c in range(hsf_ref.shape[0]):
        hsf_ref[c] = hs[:, c * 128:(c + 1) * 128]


def _tc_last_body(dinv_ref, h_ref, aggf_ref, out_ref):
    dinv = dinv_ref[...]
    agg = jnp.concatenate(
        [aggf_ref[c] for c in range(aggf_ref.shape[0])], axis=1)
    out_ref[...] = h_ref[...] - dinv * agg


def _dinv_spec():
    return pl.BlockSpec((RB, 1), lambda i: (i, 0))


def _rows_spec(d):
    return pl.BlockSpec((RB, d), lambda i: (i, 0))


def _full_spec(r, c):
    return pl.BlockSpec((r, c), lambda i: (0, 0))


def _chunk_spec(c):
    return pl.BlockSpec((c, RB, 128), lambda i: (0, i, 0))


def _tc_first(dinv, x, w, b, cout):
    d_in, d_out = w.shape
    return pl.pallas_call(
        _tc_first_body,
        grid=(GRID,),
        in_specs=[
            _dinv_spec(), _rows_spec(d_in),
            _full_spec(d_in, d_out), _full_spec(1, d_out),
        ],
        out_specs=[_rows_spec(d_out), _chunk_spec(cout)],
        out_shape=[
            jax.ShapeDtypeStruct((N, d_out), jnp.float32),
            jax.ShapeDtypeStruct((cout, NP, 128), jnp.float32),
        ],
    )(dinv, x, w, b)


def _tc_mid(dinv, h, aggf, w, b, cout):
    d_in, d_out = w.shape
    cin = aggf.shape[0]
    return pl.pallas_call(
        _tc_mid_body,
        grid=(GRID,),
        in_specs=[
            _dinv_spec(), _rows_spec(d_in), _chunk_spec(cin),
            _full_spec(d_in, d_out), _full_spec(1, d_out),
        ],
        out_specs=[_rows_spec(d_out), _chunk_spec(cout)],
        out_shape=[
            jax.ShapeDtypeStruct((N, d_out), jnp.float32),
            jax.ShapeDtypeStruct((cout, NP, 128), jnp.float32),
        ],
    )(dinv, h, aggf, w, b)


def _tc_last(dinv, h, aggf):
    d = h.shape[1]
    cin = aggf.shape[0]
    return pl.pallas_call(
        _tc_last_body,
        grid=(GRID,),
        in_specs=[_dinv_spec(), _rows_spec(d), _chunk_spec(cin)],
        out_specs=_rows_spec(d),
        out_shape=jax.ShapeDtypeStruct((N, d), jnp.float32),
    )(dinv, h, aggf)


def kernel(x, edge_index, W1, b1, W2, b2, W3, b3):
    row = edge_index[0]
    col = edge_index[1]
    parts = _deg_call(row)
    dinv = _tc_dinv(parts)

    w1b = W1.astype(jnp.bfloat16)
    h1, xsf = _tc_first(dinv, x, w1b, b1.reshape(1, -1), IN_DIM // 128)
    aggx = _make_spmm(2)(xsf.reshape(-1, 128), row, col)
    sparts = _svec_call(dinv.reshape(-1), row, col)
    svec = _tc_svec(sparts, dinv)

    h2, hs2 = _tc_layer2(
        dinv, svec, h1, aggx.reshape(-1, NP, 128),
        w1b, b1.reshape(1, -1), W2.astype(jnp.bfloat16),
        b2.reshape(1, -1), HID // 128)
    agg2 = _spmm4(hs2.reshape(-1, 128), row, col).reshape(-1, NP, 128)

    h3, hs3 = _tc_mid(dinv, h2, agg2, W3.astype(jnp.bfloat16),
                      b3.reshape(1, -1), OUT_DIM // 128)
    agg3 = _spmm2(hs3.reshape(-1, 128), row, col).reshape(-1, NP, 128)

    return _tc_last(dinv, h3, agg3)


# speedup vs baseline: 1.1200x; 1.1117x over previous
import functools

import jax
import jax.numpy as jnp
from jax import lax
from jax.experimental import pallas as pl
from jax.experimental.pallas import tpu as pltpu
from jax.experimental.pallas import tpu_sc as plsc

N = 10000
E = 160000
IN_DIM = 256
HID = 512
OUT_DIM = 256

NC = 2
NS = 16
NW = NC * NS
L = 16

RB = 400
GRID = N // RB

EPW = E // NW
DEG_BATCHES = (EPW + L - 1) // L
EPW_PAD = DEG_BATCHES * L


def _deg_body(row_hbm, parts_hbm, rowv, degv):
    wid = lax.axis_index("s") * NC + lax.axis_index("c")
    pltpu.sync_copy(row_hbm.at[pl.ds(wid * EPW, EPW)], rowv.at[pl.ds(0, EPW)])

    def zero_step(i, _):
        degv[pl.ds(i * L, L)] = jnp.zeros((L,), jnp.float32)
        return 0

    lax.fori_loop(0, N // L, zero_step, 0)

    ones = jnp.ones((L,), jnp.float32)
    lane = lax.iota(jnp.int32, L)

    def step(j, _):
        idx = rowv[pl.ds(j * L, L)]
        mask = (j * L + lane) < EPW
        plsc.addupdate_scatter(degv, [idx], ones, mask=mask)
        return 0

    lax.fori_loop(0, DEG_BATCHES, step, 0)
    pltpu.sync_copy(degv, parts_hbm.at[pl.ds(wid * N, N)])


@functools.lru_cache(maxsize=None)
def _deg_kernel():
    return pl.kernel(
        _deg_body,
        out_type=jax.ShapeDtypeStruct((NW * N,), jnp.float32),
        mesh=plsc.VectorSubcoreMesh(
            core_axis_name="c", subcore_axis_name="s",
            num_cores=NC, num_subcores=NS),
        scratch_types=[
            pltpu.VMEM((EPW_PAD,), jnp.int32),
            pltpu.VMEM((N,), jnp.float32),
        ],
        compiler_params=pltpu.CompilerParams(needs_layout_passes=False),
    )


def _deg_call(row_flat):
    return _deg_kernel()(row_flat).reshape(NW, N)

K = 80
EPS = E // NS
NB = EPS // K
NP = 10240
RPT = NP // NS
ZR = 40


def _spmm_body(cpc, hs_hbm, rows_hbm, cols_hbm, zeros_hbm, agg_hbm,
               rowb0, rowb1, rowb2, rowb3, colb0, colb1, colb2, colb3,
               gbuf0, gbuf1, gbuf2, gbuf3, acc,
               isem0, isem1, isem2, isem3,
               gsem0, gsem1, gsem2, gsem3,
               ssem0, ssem1, ssem2, ssem3):
    cid = lax.axis_index("c")
    sid = lax.axis_index("s")
    ibase = sid * EPS
    rowb = (rowb0, rowb1, rowb2, rowb3)
    colb = (colb0, colb1, colb2, colb3)
    gbuf = (gbuf0, gbuf1, gbuf2, gbuf3)
    isem = (isem0, isem1, isem2, isem3)
    gsem = (gsem0, gsem1, gsem2, gsem3)
    ssem = (ssem0, ssem1, ssem2, ssem3)

    def start_idx(j, r):
        pltpu.async_copy(rows_hbm.at[pl.ds(ibase + j * K, K)],
                         rowb[r], isem[r])
        pltpu.async_copy(cols_hbm.at[pl.ds(ibase + j * K, K)],
                         colb[r], isem[r])

    def wait_idx(r):
        pltpu.make_async_copy(rows_hbm.at[pl.ds(0, K)],
                              rowb[r], isem[r]).wait()
        pltpu.make_async_copy(cols_hbm.at[pl.ds(0, K)],
                              colb[r], isem[r]).wait()

    def add_off_and_gather(offv, r):
        def add_step(i, _):
            colb[r][pl.ds(i * L, L)] = colb[r][pl.ds(i * L, L)] + offv
            return 0

        lax.fori_loop(0, K // L, add_step, 0)
        pltpu.async_copy(hs_hbm.at[colb[r]], gbuf[r], gsem[r])

    def wait_gather(r):
        pltpu.make_async_copy(hs_hbm.at[pl.ds(0, K)],
                              gbuf[r], gsem[r]).wait()

    def start_scatter(r):
        pltpu.async_copy(gbuf[r], acc.at[rowb[r]], ssem[r], add=True)

    def wait_scatter(r):
        pltpu.make_async_copy(hs_hbm.at[pl.ds(0, K)],
                              gbuf[r], ssem[r]).wait()

    for l in range(cpc):
        chunk = cid * cpc + l
        offv = chunk * NP

        pltpu.sync_copy(zeros_hbm, acc.at[pl.ds(sid * RPT, RPT)])
        plsc.subcore_barrier()

        start_idx(0, 0)
        start_idx(1, 1)
        wait_idx(0)
        add_off_and_gather(offv, 0)

        def batch_step(j, _):
            @pl.when(j >= 2)
            def _():
                for r in range(4):
                    @pl.when((j - 2) % 4 == r)
                    def _(r=r):
                        wait_scatter(r)

            @pl.when(j + 2 < NB)
            def _():
                for r in range(4):
                    @pl.when((j + 2) % 4 == r)
                    def _(r=r):
                        start_idx(j + 2, r)

            @pl.when(j + 1 < NB)
            def _():
                for r in range(4):
                    @pl.when((j + 1) % 4 == r)
                    def _(r=r):
                        wait_idx(r)
                        add_off_and_gather(offv, r)

            for r in range(4):
                @pl.when(j % 4 == r)
                def _(r=r):
                    wait_gather(r)
                    start_scatter(r)

            return 0

        lax.fori_loop(0, NB, batch_step, 0)

        for r in range(4):
            @pl.when((NB - 2) % 4 == r)
            def _(r=r):
                wait_scatter(r)

            @pl.when((NB - 1) % 4 == r)
            def _(r=r):
                wait_scatter(r)

        plsc.subcore_barrier()
        pltpu.sync_copy(
            acc.at[pl.ds(sid * RPT, RPT)],
            agg_hbm.at[pl.ds(offv + sid * RPT, RPT)],
        )
        plsc.subcore_barrier()


def _svec_body(dinv_hbm, rows_hbm, cols_hbm, sparts_hbm,
               rowv, colv, dinvv, sacc):
    wid = lax.axis_index("s") * NC + lax.axis_index("c")
    rowv[pl.ds(EPW_PAD - L, L)] = jnp.zeros((L,), jnp.int32)
    colv[pl.ds(EPW_PAD - L, L)] = jnp.zeros((L,), jnp.int32)
    pltpu.sync_copy(rows_hbm.at[pl.ds(wid * EPW, EPW)], rowv.at[pl.ds(0, EPW)])
    pltpu.sync_copy(cols_hbm.at[pl.ds(wid * EPW, EPW)], colv.at[pl.ds(0, EPW)])
    pltpu.sync_copy(dinv_hbm, dinvv)

    def zero_step(i, _):
        sacc[pl.ds(i * L, L)] = jnp.zeros((L,), jnp.float32)
        return 0

    lax.fori_loop(0, N // L, zero_step, 0)

    lane = lax.iota(jnp.int32, L)

    def step(j, _):
        idx_c = colv[pl.ds(j * L, L)]
        idx_r = rowv[pl.ds(j * L, L)]
        mask = (j * L + lane) < EPW
        val = plsc.load_gather(dinvv, [idx_c], mask=mask)
        plsc.addupdate_scatter(sacc, [idx_r], val, mask=mask)
        return 0

    lax.fori_loop(0, DEG_BATCHES, step, 0)
    pltpu.sync_copy(sacc, sparts_hbm.at[pl.ds(wid * N, N)])


@functools.lru_cache(maxsize=None)
def _svec_kernel():
    return pl.kernel(
        _svec_body,
        out_type=jax.ShapeDtypeStruct((NW * N,), jnp.float32),
        mesh=plsc.VectorSubcoreMesh(
            core_axis_name="c", subcore_axis_name="s",
            num_cores=NC, num_subcores=NS),
        scratch_types=[
            pltpu.VMEM((EPW_PAD,), jnp.int32),
            pltpu.VMEM((EPW_PAD,), jnp.int32),
            pltpu.VMEM((N,), jnp.float32),
            pltpu.VMEM((N,), jnp.float32),
        ],
        compiler_params=pltpu.CompilerParams(needs_layout_passes=False),
    )


def _svec_call(dinv_vec, row, col):
    return _svec_kernel()(dinv_vec, row, col).reshape(NW, N)


@functools.lru_cache(maxsize=None)
def _make_spmm(c_chunks):
    cpc = c_chunks // NC
    return pl.kernel(
        functools.partial(_spmm_body, cpc),
        out_type=jax.ShapeDtypeStruct((c_chunks * NP, 128), jnp.float32),
        mesh=plsc.VectorSubcoreMesh(
            core_axis_name="c", subcore_axis_name="s",
            num_cores=NC, num_subcores=NS),
        scratch_types=(
            [pltpu.VMEM((K,), jnp.int32) for _ in range(8)]
            + [pltpu.VMEM((K, 128), jnp.float32) for _ in range(4)]
            + [pltpu.VMEM_SHARED((NP, 128), jnp.float32)]
            + [pltpu.SemaphoreType.DMA for _ in range(12)]
        ),
        compiler_params=pltpu.CompilerParams(needs_layout_passes=False),
    )


def _spmm4(hs_flat, row, col, zeros):
    return _make_spmm(4)(hs_flat, row, col, zeros)


def _spmm2(hs_flat, row, col, zeros):
    return _make_spmm(2)(hs_flat, row, col, zeros)


def _tc_dinv_body(parts_ref, dinv_ref):
    deg = jnp.sum(parts_ref[...], axis=0)
    dinv_ref[...] = jnp.where(
        deg > 0, lax.rsqrt(jnp.maximum(deg, 1e-12)), 0.0)


def _tc_dinv(parts):
    return pl.pallas_call(
        _tc_dinv_body,
        out_shape=jax.ShapeDtypeStruct((N,), jnp.float32),
    )(parts).reshape(N, 1)


def _tc_first_body(dinv_ref, x_ref, w_ref, b_ref, h_ref, xsf_ref):
    dinv = dinv_ref[...]
    h = jnp.dot(x_ref[...].astype(jnp.bfloat16), w_ref[...],
                preferred_element_type=jnp.float32)
    h_ref[...] = h + b_ref[...]
    xs = x_ref[...] * dinv
    for c in range(xsf_ref.shape[0]):
        xsf_ref[c] = xs[:, c * 128:(c + 1) * 128]


def _tc_svec_body(sparts_ref, dinv_ref, s_ref):
    sp = jnp.sum(sparts_ref[...], axis=0)
    s_ref[...] = sp * dinv_ref[...][:, 0]


def _tc_svec(sparts, dinv):
    return pl.pallas_call(
        _tc_svec_body,
        out_shape=jax.ShapeDtypeStruct((N,), jnp.float32),
    )(sparts, dinv).reshape(N, 1)


def _tc_layer2_body(dinv_ref, s_ref, h1_ref, aggxf_ref, w1_ref, b1_ref,
                    w2_ref, b2_ref, ho_ref, hsf_ref):
    dinv = dinv_ref[...]
    aggx = jnp.concatenate(
        [aggxf_ref[c] for c in range(aggxf_ref.shape[0])], axis=1)
    corr = jnp.dot((dinv * aggx).astype(jnp.bfloat16), w1_ref[...],
                   preferred_element_type=jnp.float32)
    t = jnp.maximum(h1_ref[...] - corr - s_ref[...] * b1_ref[...], 0.0)
    h = jnp.dot(t.astype(jnp.bfloat16), w2_ref[...],
                preferred_element_type=jnp.float32)
    h = h + b2_ref[...]
    ho_ref[...] = h
    hs = h * dinv
    for c in range(hsf_ref.shape[0]):
        hsf_ref[c] = hs[:, c * 128:(c + 1) * 128]


def _tc_layer2(dinv, svec, h1, aggxf, w1, b1, w2, b2, cout):
    d1_in, d1_out = w1.shape
    d2_out = w2.shape[1]
    cin = aggxf.shape[0]
    return pl.pallas_call(
        _tc_layer2_body,
        grid=(GRID,),
        in_specs=[
            _dinv_spec(), _dinv_spec(), _rows_spec(d1_out), _chunk_spec(cin),
            _full_spec(d1_in, d1_out), _full_spec(1, d1_out),
            _full_spec(d1_out, d2_out), _full_spec(1, d2_out),
        ],
        out_specs=[_rows_spec(d2_out), _chunk_spec(cout)],
        out_shape=[
            jax.ShapeDtypeStruct((N, d2_out), jnp.float32),
            jax.ShapeDtypeStruct((cout, NP, 128), jnp.float32),
        ],
    )(dinv, svec, h1, aggxf, w1, b1, w2, b2)


def _tc_mid_body(dinv_ref, h_ref, aggf_ref, w_ref, b_ref, ho_ref, hsf_ref):
    dinv = dinv_ref[...]
    agg = jnp.concatenate(
        [aggf_ref[c] for c in range(aggf_ref.shape[0])], axis=1)
    t = jnp.maximum(h_ref[...] - dinv * agg, 0.0)
    h = jnp.dot(t.astype(jnp.bfloat16), w_ref[...],
                preferred_element_type=jnp.float32)
    h = h + b_ref[...]
    ho_ref[...] = h
    hs = h * dinv
    for c in range(hsf_ref.shape[0]):
        hsf_ref[c] = hs[:, c * 128:(c + 1) * 128]


def _tc_last_body(dinv_ref, h_ref, aggf_ref, out_ref):
    dinv = dinv_ref[...]
    agg = jnp.concatenate(
        [aggf_ref[c] for c in range(aggf_ref.shape[0])], axis=1)
    out_ref[...] = h_ref[...] - dinv * agg


def _dinv_spec():
    return pl.BlockSpec((RB, 1), lambda i: (i, 0))


def _rows_spec(d):
    return pl.BlockSpec((RB, d), lambda i: (i, 0))


def _full_spec(r, c):
    return pl.BlockSpec((r, c), lambda i: (0, 0))


def _chunk_spec(c):
    return pl.BlockSpec((c, RB, 128), lambda i: (0, i, 0))


def _tc_first(dinv, x, w, b, cout):
    d_in, d_out = w.shape
    return pl.pallas_call(
        _tc_first_body,
        grid=(GRID,),
        in_specs=[
            _dinv_spec(), _rows_spec(d_in),
            _full_spec(d_in, d_out), _full_spec(1, d_out),
        ],
        out_specs=[_rows_spec(d_out), _chunk_spec(cout)],
        out_shape=[
            jax.ShapeDtypeStruct((N, d_out), jnp.float32),
            jax.ShapeDtypeStruct((cout, NP, 128), jnp.float32),
        ],
    )(dinv, x, w, b)


def _tc_mid(dinv, h, aggf, w, b, cout):
    d_in, d_out = w.shape
    cin = aggf.shape[0]
    return pl.pallas_call(
        _tc_mid_body,
        grid=(GRID,),
        in_specs=[
            _dinv_spec(), _rows_spec(d_in), _chunk_spec(cin),
            _full_spec(d_in, d_out), _full_spec(1, d_out),
        ],
        out_specs=[_rows_spec(d_out), _chunk_spec(cout)],
        out_shape=[
            jax.ShapeDtypeStruct((N, d_out), jnp.float32),
            jax.ShapeDtypeStruct((cout, NP, 128), jnp.float32),
        ],
    )(dinv, h, aggf, w, b)


def _tc_last(dinv, h, aggf):
    d = h.shape[1]
    cin = aggf.shape[0]
    return pl.pallas_call(
        _tc_last_body,
        grid=(GRID,),
        in_specs=[_dinv_spec(), _rows_spec(d), _chunk_spec(cin)],
        out_specs=_rows_spec(d),
        out_shape=jax.ShapeDtypeStruct((N, d), jnp.float32),
    )(dinv, h, aggf)


def kernel(x, edge_index, W1, b1, W2, b2, W3, b3):
    row = edge_index[0]
    col = edge_index[1]
    zeros = jnp.zeros((RPT, 128), jnp.float32)

    parts = _deg_call(row)
    dinv = _tc_dinv(parts)

    w1b = W1.astype(jnp.bfloat16)
    h1, xsf = _tc_first(dinv, x, w1b, b1.reshape(1, -1), IN_DIM // 128)
    aggx = _make_spmm(2)(xsf.reshape(-1, 128), row, col, zeros)
    sparts = _svec_call(dinv.reshape(-1), row, col)
    svec = _tc_svec(sparts, dinv)

    h2, hs2 = _tc_layer2(
        dinv, svec, h1, aggx.reshape(-1, NP, 128),
        w1b, b1.reshape(1, -1), W2.astype(jnp.bfloat16),
        b2.reshape(1, -1), HID // 128)
    agg2 = _spmm4(hs2.reshape(-1, 128), row, col, zeros).reshape(-1, NP, 128)

    h3, hs3 = _tc_mid(dinv, h2, agg2, W3.astype(jnp.bfloat16),
                      b3.reshape(1, -1), OUT_DIM // 128)
    agg3 = _spmm2(hs3.reshape(-1, 128), row, col, zeros).reshape(-1, NP, 128)

    return _tc_last(dinv, h3, agg3)


# speedup vs baseline: 1.1326x; 1.0113x over previous
import functools

import jax
import jax.numpy as jnp
from jax import lax
from jax.experimental import pallas as pl
from jax.experimental.pallas import tpu as pltpu
from jax.experimental.pallas import tpu_sc as plsc

N = 10000
E = 160000
IN_DIM = 256
HID = 512
OUT_DIM = 256

NC = 2
NS = 16
NW = NC * NS
L = 16

RB = 400
GRID = N // RB

EPW = E // NW
DEG_BATCHES = (EPW + L - 1) // L
EPW_PAD = DEG_BATCHES * L


def _deg_body(row_hbm, parts_hbm, rowv, degv):
    wid = lax.axis_index("s") * NC + lax.axis_index("c")
    pltpu.sync_copy(row_hbm.at[pl.ds(wid * EPW, EPW)], rowv.at[pl.ds(0, EPW)])

    def zero_step(i, _):
        degv[pl.ds(i * L, L)] = jnp.zeros((L,), jnp.float32)
        return 0

    lax.fori_loop(0, N // L, zero_step, 0)

    ones = jnp.ones((L,), jnp.float32)
    lane = lax.iota(jnp.int32, L)

    def step(j, _):
        idx = rowv[pl.ds(j * L, L)]
        mask = (j * L + lane) < EPW
        plsc.addupdate_scatter(degv, [idx], ones, mask=mask)
        return 0

    lax.fori_loop(0, DEG_BATCHES, step, 0)
    pltpu.sync_copy(degv, parts_hbm.at[pl.ds(wid * N, N)])


@functools.lru_cache(maxsize=None)
def _deg_kernel():
    return pl.kernel(
        _deg_body,
        out_type=jax.ShapeDtypeStruct((NW * N,), jnp.float32),
        mesh=plsc.VectorSubcoreMesh(
            core_axis_name="c", subcore_axis_name="s",
            num_cores=NC, num_subcores=NS),
        scratch_types=[
            pltpu.VMEM((EPW_PAD,), jnp.int32),
            pltpu.VMEM((N,), jnp.float32),
        ],
        compiler_params=pltpu.CompilerParams(needs_layout_passes=False),
    )


def _deg_call(row_flat):
    return _deg_kernel()(row_flat).reshape(NW, N)

K = 80
EPS = E // NS
NB = EPS // K
NP = 10240
RPT = NP // NS
ZR = 40


def _spmm_body(cpc, hs_hbm, rows_hbm, cols_hbm, zeros_hbm, agg_hbm,
               rowb0, rowb1, rowb2, rowb3, colb0, colb1, colb2, colb3,
               gbuf0, gbuf1, gbuf2, gbuf3, acc,
               isem0, isem1, isem2, isem3,
               gsem0, gsem1, gsem2, gsem3,
               ssem0, ssem1, ssem2, ssem3):
    cid = lax.axis_index("c")
    sid = lax.axis_index("s")
    ibase = sid * EPS
    rowb = (rowb0, rowb1, rowb2, rowb3)
    colb = (colb0, colb1, colb2, colb3)
    gbuf = (gbuf0, gbuf1, gbuf2, gbuf3)
    isem = (isem0, isem1, isem2, isem3)
    gsem = (gsem0, gsem1, gsem2, gsem3)
    ssem = (ssem0, ssem1, ssem2, ssem3)

    def start_idx(j, r):
        pltpu.async_copy(rows_hbm.at[pl.ds(ibase + j * K, K)],
                         rowb[r], isem[r])
        pltpu.async_copy(cols_hbm.at[pl.ds(ibase + j * K, K)],
                         colb[r], isem[r])

    def wait_idx(r):
        pltpu.make_async_copy(rows_hbm.at[pl.ds(0, K)],
                              rowb[r], isem[r]).wait()
        pltpu.make_async_copy(cols_hbm.at[pl.ds(0, K)],
                              colb[r], isem[r]).wait()

    def add_off_and_gather(offv, r):
        def add_step(i, _):
            colb[r][pl.ds(i * L, L)] = colb[r][pl.ds(i * L, L)] + offv
            return 0

        lax.fori_loop(0, K // L, add_step, 0)
        pltpu.async_copy(hs_hbm.at[colb[r]], gbuf[r], gsem[r])

    def wait_gather(r):
        pltpu.make_async_copy(hs_hbm.at[pl.ds(0, K)],
                              gbuf[r], gsem[r]).wait()

    def start_scatter(r):
        pltpu.async_copy(gbuf[r], acc.at[rowb[r]], ssem[r], add=True)

    def wait_scatter(r):
        pltpu.make_async_copy(hs_hbm.at[pl.ds(0, K)],
                              gbuf[r], ssem[r]).wait()

    for l in range(cpc):
        chunk = cid * cpc + l
        offv = chunk * NP

        pltpu.sync_copy(zeros_hbm, acc.at[pl.ds(sid * RPT, RPT)])
        plsc.subcore_barrier()

        start_idx(0, 0)
        start_idx(1, 1)
        wait_idx(0)
        add_off_and_gather(offv, 0)

        def batch_step(j, _):
            @pl.when(j >= 2)
            def _():
                for r in range(4):
                    @pl.when((j - 2) % 4 == r)
                    def _(r=r):
                        wait_scatter(r)

            @pl.when(j + 2 < NB)
            def _():
                for r in range(4):
                    @pl.when((j + 2) % 4 == r)
                    def _(r=r):
                        start_idx(j + 2, r)

            @pl.when(j + 1 < NB)
            def _():
                for r in range(4):
                    @pl.when((j + 1) % 4 == r)
                    def _(r=r):
                        wait_idx(r)
                        add_off_and_gather(offv, r)

            for r in range(4):
                @pl.when(j % 4 == r)
                def _(r=r):
                    wait_gather(r)
                    start_scatter(r)

            return 0

        lax.fori_loop(0, NB, batch_step, 0)

        for r in range(4):
            @pl.when((NB - 2) % 4 == r)
            def _(r=r):
                wait_scatter(r)

            @pl.when((NB - 1) % 4 == r)
            def _(r=r):
                wait_scatter(r)

        plsc.subcore_barrier()
        pltpu.sync_copy(
            acc.at[pl.ds(sid * RPT, RPT)],
            agg_hbm.at[pl.ds(offv + sid * RPT, RPT)],
        )
        plsc.subcore_barrier()


def _svec_body(dinv_hbm, rows_hbm, cols_hbm, sparts_hbm,
               rowv, colv, dinvv, sacc):
    wid = lax.axis_index("s") * NC + lax.axis_index("c")
    rowv[pl.ds(EPW_PAD - L, L)] = jnp.zeros((L,), jnp.int32)
    colv[pl.ds(EPW_PAD - L, L)] = jnp.zeros((L,), jnp.int32)
    pltpu.sync_copy(rows_hbm.at[pl.ds(wid * EPW, EPW)], rowv.at[pl.ds(0, EPW)])
    pltpu.sync_copy(cols_hbm.at[pl.ds(wid * EPW, EPW)], colv.at[pl.ds(0, EPW)])
    pltpu.sync_copy(dinv_hbm, dinvv)

    def zero_step(i, _):
        sacc[pl.ds(i * L, L)] = jnp.zeros((L,), jnp.float32)
        return 0

    lax.fori_loop(0, N // L, zero_step, 0)

    lane = lax.iota(jnp.int32, L)

    def step(j, _):
        idx_c = colv[pl.ds(j * L, L)]
        idx_r = rowv[pl.ds(j * L, L)]
        mask = (j * L + lane) < EPW
        val = plsc.load_gather(dinvv, [idx_c], mask=mask)
        plsc.addupdate_scatter(sacc, [idx_r], val, mask=mask)
        return 0

    lax.fori_loop(0, DEG_BATCHES, step, 0)
    pltpu.sync_copy(sacc, sparts_hbm.at[pl.ds(wid * N, N)])


@functools.lru_cache(maxsize=None)
def _svec_kernel():
    return pl.kernel(
        _svec_body,
        out_type=jax.ShapeDtypeStruct((NW * N,), jnp.float32),
        mesh=plsc.VectorSubcoreMesh(
            core_axis_name="c", subcore_axis_name="s",
            num_cores=NC, num_subcores=NS),
        scratch_types=[
            pltpu.VMEM((EPW_PAD,), jnp.int32),
            pltpu.VMEM((EPW_PAD,), jnp.int32),
            pltpu.VMEM((N,), jnp.float32),
            pltpu.VMEM((N,), jnp.float32),
        ],
        compiler_params=pltpu.CompilerParams(needs_layout_passes=False),
    )


def _svec_call(dinv_vec, row, col):
    return _svec_kernel()(dinv_vec, row, col).reshape(NW, N)


@functools.lru_cache(maxsize=None)
def _make_spmm(c_chunks):
    cpc = c_chunks // NC
    return pl.kernel(
        functools.partial(_spmm_body, cpc),
        out_type=jax.ShapeDtypeStruct((c_chunks * NP, 128), jnp.float32),
        mesh=plsc.VectorSubcoreMesh(
            core_axis_name="c", subcore_axis_name="s",
            num_cores=NC, num_subcores=NS),
        scratch_types=(
            [pltpu.VMEM((K,), jnp.int32) for _ in range(8)]
            + [pltpu.VMEM((K, 128), jnp.float32) for _ in range(4)]
            + [pltpu.VMEM_SHARED((NP, 128), jnp.float32)]
            + [pltpu.SemaphoreType.DMA for _ in range(12)]
        ),
        compiler_params=pltpu.CompilerParams(needs_layout_passes=False),
    )


def _spmm4(hs_flat, row, col, zeros):
    return _make_spmm(4)(hs_flat, row, col, zeros)


def _spmm2(hs_flat, row, col, zeros):
    return _make_spmm(2)(hs_flat, row, col, zeros)


def _tc_dinv_body(parts_ref, dinv_ref):
    deg = jnp.sum(parts_ref[...], axis=0)
    dinv_ref[...] = jnp.where(
        deg > 0, lax.rsqrt(jnp.maximum(deg, 1e-12)), 0.0)


def _tc_dinv(parts):
    return pl.pallas_call(
        _tc_dinv_body,
        out_shape=jax.ShapeDtypeStruct((N,), jnp.float32),
    )(parts).reshape(N, 1)


def _tc_scale_body(dinv_ref, x_ref, xsf_ref):
    xs = x_ref[...] * dinv_ref[...]
    for c in range(xsf_ref.shape[0]):
        xsf_ref[c] = xs[:, c * 128:(c + 1) * 128]


def _tc_scale(dinv, x):
    d = x.shape[1]
    cout = d // 128
    return pl.pallas_call(
        _tc_scale_body,
        grid=(GRID,),
        in_specs=[_dinv_spec(), _rows_spec(d)],
        out_specs=_chunk_spec(cout),
        out_shape=jax.ShapeDtypeStruct((cout, NP, 128), jnp.float32),
    )(dinv, x)


def _tc_h1_body(x_ref, w_ref, b_ref, h_ref):
    h = jnp.dot(x_ref[...].astype(jnp.bfloat16), w_ref[...],
                preferred_element_type=jnp.float32)
    h_ref[...] = h + b_ref[...]


def _tc_h1(x, w, b):
    d_in, d_out = w.shape
    return pl.pallas_call(
        _tc_h1_body,
        grid=(GRID,),
        in_specs=[_rows_spec(d_in), _full_spec(d_in, d_out),
                  _full_spec(1, d_out)],
        out_specs=_rows_spec(d_out),
        out_shape=jax.ShapeDtypeStruct((N, d_out), jnp.float32),
    )(x, w, b)


def _tc_svec_body(sparts_ref, dinv_ref, s_ref):
    sp = jnp.sum(sparts_ref[...], axis=0)
    s_ref[...] = sp * dinv_ref[...][:, 0]


def _tc_svec(sparts, dinv):
    return pl.pallas_call(
        _tc_svec_body,
        out_shape=jax.ShapeDtypeStruct((N,), jnp.float32),
    )(sparts, dinv).reshape(N, 1)


def _tc_layer2_body(dinv_ref, s_ref, h1_ref, aggxf_ref, w1_ref, b1_ref,
                    w2_ref, b2_ref, ho_ref, hsf_ref):
    dinv = dinv_ref[...]
    aggx = jnp.concatenate(
        [aggxf_ref[c] for c in range(aggxf_ref.shape[0])], axis=1)
    corr = jnp.dot((dinv * aggx).astype(jnp.bfloat16), w1_ref[...],
                   preferred_element_type=jnp.float32)
    t = jnp.maximum(h1_ref[...] - corr - s_ref[...] * b1_ref[...], 0.0)
    h = jnp.dot(t.astype(jnp.bfloat16), w2_ref[...],
                preferred_element_type=jnp.float32)
    h = h + b2_ref[...]
    ho_ref[...] = h
    hs = h * dinv
    for c in range(hsf_ref.shape[0]):
        hsf_ref[c] = hs[:, c * 128:(c + 1) * 128]


def _tc_layer2(dinv, svec, h1, aggxf, w1, b1, w2, b2, cout):
    d1_in, d1_out = w1.shape
    d2_out = w2.shape[1]
    cin = aggxf.shape[0]
    return pl.pallas_call(
        _tc_layer2_body,
        grid=(GRID,),
        in_specs=[
            _dinv_spec(), _dinv_spec(), _rows_spec(d1_out), _chunk_spec(cin),
            _full_spec(d1_in, d1_out), _full_spec(1, d1_out),
            _full_spec(d1_out, d2_out), _full_spec(1, d2_out),
        ],
        out_specs=[_rows_spec(d2_out), _chunk_spec(cout)],
        out_shape=[
            jax.ShapeDtypeStruct((N, d2_out), jnp.float32),
            jax.ShapeDtypeStruct((cout, NP, 128), jnp.float32),
        ],
    )(dinv, svec, h1, aggxf, w1, b1, w2, b2)


def _tc_mid_body(dinv_ref, h_ref, aggf_ref, w_ref, b_ref, ho_ref, hsf_ref):
    dinv = dinv_ref[...]
    agg = jnp.concatenate(
        [aggf_ref[c] for c in range(aggf_ref.shape[0])], axis=1)
    t = jnp.maximum(h_ref[...] - dinv * agg, 0.0)
    h = jnp.dot(t.astype(jnp.bfloat16), w_ref[...],
                preferred_element_type=jnp.float32)
    h = h + b_ref[...]
    ho_ref[...] = h
    hs = h * dinv
    for c in range(hsf_ref.shape[0]):
        hsf_ref[c] = hs[:, c * 128:(c + 1) * 128]


def _tc_last_body(dinv_ref, h_ref, aggf_ref, out_ref):
    dinv = dinv_ref[...]
    agg = jnp.concatenate(
        [aggf_ref[c] for c in range(aggf_ref.shape[0])], axis=1)
    out_ref[...] = h_ref[...] - dinv * agg


def _dinv_spec():
    return pl.BlockSpec((RB, 1), lambda i: (i, 0))


def _rows_spec(d):
    return pl.BlockSpec((RB, d), lambda i: (i, 0))


def _full_spec(r, c):
    return pl.BlockSpec((r, c), lambda i: (0, 0))


def _chunk_spec(c):
    return pl.BlockSpec((c, RB, 128), lambda i: (0, i, 0))


def _tc_mid(dinv, h, aggf, w, b, cout):
    d_in, d_out = w.shape
    cin = aggf.shape[0]
    return pl.pallas_call(
        _tc_mid_body,
        grid=(GRID,),
        in_specs=[
            _dinv_spec(), _rows_spec(d_in), _chunk_spec(cin),
            _full_spec(d_in, d_out), _full_spec(1, d_out),
        ],
        out_specs=[_rows_spec(d_out), _chunk_spec(cout)],
        out_shape=[
            jax.ShapeDtypeStruct((N, d_out), jnp.float32),
            jax.ShapeDtypeStruct((cout, NP, 128), jnp.float32),
        ],
    )(dinv, h, aggf, w, b)


def _tc_last(dinv, h, aggf):
    d = h.shape[1]
    cin = aggf.shape[0]
    return pl.pallas_call(
        _tc_last_body,
        grid=(GRID,),
        in_specs=[_dinv_spec(), _rows_spec(d), _chunk_spec(cin)],
        out_specs=_rows_spec(d),
        out_shape=jax.ShapeDtypeStruct((N, d), jnp.float32),
    )(dinv, h, aggf)


def kernel(x, edge_index, W1, b1, W2, b2, W3, b3):
    row = edge_index[0]
    col = edge_index[1]
    zeros = jnp.zeros((RPT, 128), jnp.float32)

    parts = _deg_call(row)
    dinv = _tc_dinv(parts)

    w1b = W1.astype(jnp.bfloat16)
    xsf = _tc_scale(dinv, x)
    aggx = _make_spmm(2)(xsf.reshape(-1, 128), row, col, zeros)
    sparts = _svec_call(dinv.reshape(-1), row, col)
    h1 = _tc_h1(x, w1b, b1.reshape(1, -1))
    svec = _tc_svec(sparts, dinv)

    h2, hs2 = _tc_layer2(
        dinv, svec, h1, aggx.reshape(-1, NP, 128),
        w1b, b1.reshape(1, -1), W2.astype(jnp.bfloat16),
        b2.reshape(1, -1), HID // 128)
    agg2 = _spmm4(hs2.reshape(-1, 128), row, col, zeros).reshape(-1, NP, 128)

    h3, hs3 = _tc_mid(dinv, h2, agg2, W3.astype(jnp.bfloat16),
                      b3.reshape(1, -1), OUT_DIM // 128)
    agg3 = _spmm2(hs3.reshape(-1, 128), row, col, zeros).reshape(-1, NP, 128)

    return _tc_last(dinv, h3, agg3)
